# Initial kernel scaffold; baseline (speedup 1.0000x reference)
import functools

import jax
import jax.numpy as jnp
from jax import lax
from jax.experimental import pallas as pl
from jax.experimental.pallas import tpu as pltpu
from jax.experimental.pallas import tpu_sc as plsc

B = 1024
S = 51
D = 128
TOTAL = B * S
NW = 32
PER_W = TOTAL // NW
CHUNK = 96
NCHUNK = PER_W // CHUNK


def _build():
    info = plsc.get_sparse_core_info()
    nc = info.num_cores
    mesh = plsc.VectorSubcoreMesh(core_axis_name="c", subcore_axis_name="s")

    @functools.partial(
        pl.kernel,
        mesh=mesh,
        out_type=jax.ShapeDtypeStruct((NW, NCHUNK, CHUNK, D), jnp.float32),
        scratch_types=[
            pltpu.VMEM((NCHUNK, CHUNK), jnp.int32),
            pltpu.VMEM((NCHUNK, CHUNK), jnp.int32),
            pltpu.VMEM((CHUNK, D), jnp.float32),
            pltpu.SemaphoreType.DMA,
        ],
    )
    def k(table_hbm, idx_hbm, msk_hbm, out_hbm, idx_v, msk_v, rows_v, sem):
        wid = lax.axis_index("s") * nc + lax.axis_index("c")
        pltpu.sync_copy(idx_hbm.at[wid], idx_v)
        pltpu.sync_copy(msk_hbm.at[wid], msk_v)
        for c in range(NCHUNK):
            pltpu.async_copy(table_hbm.at[idx_v.at[c]], rows_v, sem).wait()

            def zrow(r, carry, c=c):
                @pl.when(msk_v[c, r] != 0)
                def _():
                    z = jnp.zeros((16,), jnp.float32)
                    for j in range(D // 16):
                        rows_v[r, pl.ds(j * 16, 16)] = z

                return carry

            lax.fori_loop(0, CHUNK, zrow, 0)
            pltpu.sync_copy(rows_v, out_hbm.at[wid, c])

    return k


_k = jax.jit(_build())


def kernel(node_ids, mask, emb_table):
    idx = node_ids.astype(jnp.int32).reshape(NW, NCHUNK, CHUNK)
    msk = mask.astype(jnp.int32).reshape(NW, NCHUNK, CHUNK)
    out = _k(emb_table, idx, msk)
    return out.reshape(B, S, D)


# speedup vs baseline: 1.0317x; 1.0317x over previous
import functools

import jax
import jax.numpy as jnp
from jax import lax
from jax.experimental import pallas as pl
from jax.experimental.pallas import tpu as pltpu
from jax.experimental.pallas import tpu_sc as plsc

B = 1024
S = 51
D = 128
TOTAL = B * S
NW = 32
PER_W = TOTAL // NW
CHUNK = 96
NCHUNK = PER_W // CHUNK
L = 16


def _build():
    info = plsc.get_sparse_core_info()
    nc = info.num_cores
    mesh = plsc.VectorSubcoreMesh(core_axis_name="c", subcore_axis_name="s")

    @functools.partial(
        pl.kernel,
        mesh=mesh,
        out_type=jax.ShapeDtypeStruct((NW, NCHUNK, CHUNK, D), jnp.float32),
        scratch_types=[
            pltpu.VMEM((NCHUNK, CHUNK), jnp.int32),
            pltpu.VMEM((PER_W,), jnp.int32),
            pltpu.VMEM((CHUNK, D), jnp.float32),
            pltpu.SemaphoreType.DMA,
        ],
    )
    def k(table_hbm, idx_hbm, msk_hbm, out_hbm, idx_v, msk_v, rows_v, sem):
        wid = lax.axis_index("s") * nc + lax.axis_index("c")
        pltpu.sync_copy(idx_hbm.at[wid], idx_v)
        pltpu.sync_copy(msk_hbm.at[wid], msk_v)
        lane = lax.iota(jnp.int32, L)
        zeros = jnp.zeros((L,), jnp.float32)

        def chunk_body(c, carry):
            pltpu.async_copy(table_hbm.at[idx_v.at[c]], rows_v, sem).wait()
            for g in range(CHUNK // L):
                mv = msk_v[pl.ds(c * CHUNK + g * L, L)]
                for t in range(L):
                    r = g * L + t

                    @pl.when(mv[t] != 0)
                    def _(r=r):
                        for j in range(D // L):
                            rows_v[r, pl.ds(j * L, L)] = zeros

            pltpu.sync_copy(rows_v, out_hbm.at[wid, c])
            return carry

        lax.fori_loop(0, NCHUNK, chunk_body, 0)

    return k


_k = jax.jit(_build())


def kernel(node_ids, mask, emb_table):
    idx = node_ids.astype(jnp.int32).reshape(NW, NCHUNK, CHUNK)
    msk = mask.astype(jnp.int32).reshape(NW, PER_W)
    out = _k(emb_table, idx, msk)
    return out.reshape(B, S, D)


# speedup vs baseline: 1.1688x; 1.1329x over previous
import functools

import jax
import jax.numpy as jnp
from jax import lax
from jax.experimental import pallas as pl
from jax.experimental.pallas import tpu as pltpu
from jax.experimental.pallas import tpu_sc as plsc

B = 1024
S = 51
D = 128
TOTAL = B * S
NW = 32
PER_W = TOTAL // NW
CHUNK = 32
NCHUNK = PER_W // CHUNK
NBUF = 3
L = 16


def _build():
    info = plsc.get_sparse_core_info()
    nc = info.num_cores
    mesh = plsc.VectorSubcoreMesh(core_axis_name="c", subcore_axis_name="s")

    @functools.partial(
        pl.kernel,
        mesh=mesh,
        out_type=jax.ShapeDtypeStruct((NW, NCHUNK, CHUNK, D), jnp.float32),
        scratch_types=[
            pltpu.VMEM((NCHUNK, CHUNK), jnp.int32),
            pltpu.VMEM((PER_W,), jnp.int32),
            pltpu.VMEM((NBUF, CHUNK, D), jnp.float32),
        ]
        + [pltpu.SemaphoreType.DMA] * (2 * NBUF),
    )
    def k(table_hbm, idx_hbm, msk_hbm, out_hbm, idx_v, msk_v, rows_v, *sems):
        gsem = sems[:NBUF]
        ssem = sems[NBUF:]
        wid = lax.axis_index("s") * nc + lax.axis_index("c")
        pltpu.sync_copy(idx_hbm.at[wid], idx_v)
        for b in range(NBUF):
            pltpu.async_copy(table_hbm.at[idx_v.at[b]], rows_v.at[b], gsem[b])
        pltpu.sync_copy(msk_hbm.at[wid], msk_v)
        zeros = jnp.zeros((L,), jnp.float32)

        def body(p, carry):
            for b in range(NBUF):
                c = p * NBUF + b
                prev = (b - 1) % NBUF

                @pl.when(jnp.logical_and(c >= 1, c - 1 + NBUF < NCHUNK))
                def _(b=b, c=c, prev=prev):
                    pltpu.make_async_copy(
                        rows_v.at[prev], out_hbm.at[wid, c - 1], ssem[prev]
                    ).wait()
                    pltpu.async_copy(
                        table_hbm.at[idx_v.at[c - 1 + NBUF]],
                        rows_v.at[prev],
                        gsem[prev],
                    )

                pltpu.make_async_copy(
                    table_hbm.at[idx_v.at[c]], rows_v.at[b], gsem[b]
                ).wait()
                for g in range(CHUNK // L):
                    mv = msk_v[pl.ds(c * CHUNK + g * L, L)]
                    for t in range(L):
                        r = g * L + t

                        @pl.when(mv[t] != 0)
                        def _(r=r, b=b):
                            for j in range(D // L):
                                rows_v[b, r, pl.ds(j * L, L)] = zeros

                pltpu.async_copy(rows_v.at[b], out_hbm.at[wid, c], ssem[b])

            return carry

        lax.fori_loop(0, NCHUNK // NBUF, body, 0)
        for b in range(NBUF):
            pltpu.make_async_copy(
                rows_v.at[b], out_hbm.at[wid, NCHUNK - NBUF + b], ssem[b]
            ).wait()

    return k


_k = jax.jit(_build())


def kernel(node_ids, mask, emb_table):
    idx = node_ids.astype(jnp.int32).reshape(NW, NCHUNK, CHUNK)
    msk = mask.astype(jnp.int32).reshape(NW, PER_W)
    out = _k(emb_table, idx, msk)
    return out.reshape(B, S, D)
